# baseline (speedup 1.0000x reference)
import functools

import jax
import jax.numpy as jnp
from jax import lax
from jax.experimental import pallas as pl
from jax.experimental.pallas import tpu as pltpu
from jax.experimental.pallas import tpu_sc as plsc

BATCH = 16384
HIDDEN = 64
NUM_NODES = 1000000
NUM_RELATIONS = 1000
L = 16
PW = 2 * HIDDEN

_info = plsc.get_sparse_core_info()
NC, NS = _info.num_cores, _info.num_subcores
NW = NC * NS
BPW = BATCH // NW
CHUNK = 128
NCHUNK = BPW // CHUNK
GPC = CHUNK // L
NBUF = 2

_mesh = plsc.VectorSubcoreMesh(core_axis_name="c", subcore_axis_name="s")

_pair_bufs = [pltpu.VMEM((CHUNK, PW), jnp.float32)
              for _ in range(3 * NBUF)]


@functools.partial(
    pl.kernel,
    mesh=_mesh,
    out_type=jax.ShapeDtypeStruct((BATCH,), jnp.float32),
    compiler_params=pltpu.CompilerParams(
        needs_layout_passes=False, use_tc_tiling_on_sc=False
    ),
    scratch_types=[
        pltpu.VMEM((BPW,), jnp.int32),
        pltpu.VMEM((BPW,), jnp.int32),
        pltpu.VMEM((BPW,), jnp.int32),
        pltpu.VMEM((NCHUNK, CHUNK), jnp.int32),
        pltpu.VMEM((NCHUNK, CHUNK), jnp.int32),
        pltpu.VMEM((NCHUNK, CHUNK), jnp.int32),
        pltpu.VMEM((BPW,), jnp.float32),
    ] + _pair_bufs + [pltpu.SemaphoreType.DMA for _ in range(NBUF)],
)
def _kge_score_sc(head_hbm, rel_hbm, tail_hbm, nodep_hbm, relp_hbm, out_hbm,
                  idx_h, idx_r, idx_t, gi_h, gi_r, gi_t, scores,
                  *bufs_and_sems):
    bufs = [bufs_and_sems[3 * b:3 * b + 3] for b in range(NBUF)]
    sems = bufs_and_sems[3 * NBUF:]

    wid = lax.axis_index("s") * NC + lax.axis_index("c")
    base = wid * BPW

    pltpu.sync_copy(head_hbm.at[pl.ds(base, BPW)], idx_h)
    pltpu.sync_copy(rel_hbm.at[pl.ds(base, BPW)], idx_r)
    pltpu.sync_copy(tail_hbm.at[pl.ds(base, BPW)], idx_t)

    for raw, gi in ((idx_h, gi_h), (idx_r, gi_r), (idx_t, gi_t)):
        for c in range(NCHUNK):
            for m in range(CHUNK // L):
                v = raw[pl.ds(c * CHUNK + m * L, L)]
                gi[c, pl.ds(m * L, L)] = v >> 1

    lanes = lax.iota(jnp.int32, L)

    def fire(c, b):
        hbuf, rbuf, tbuf = bufs[b]
        pltpu.async_copy(nodep_hbm.at[gi_h.at[c]], hbuf, sems[b])
        pltpu.async_copy(relp_hbm.at[gi_r.at[c]], rbuf, sems[b])
        pltpu.async_copy(nodep_hbm.at[gi_t.at[c]], tbuf, sems[b])

    def drain_and_compute(c, b):
        hbuf, rbuf, tbuf = bufs[b]
        pltpu.make_async_copy(nodep_hbm.at[gi_h.at[c]], hbuf, sems[b]).wait()
        pltpu.make_async_copy(relp_hbm.at[gi_r.at[c]], rbuf, sems[b]).wait()
        pltpu.make_async_copy(nodep_hbm.at[gi_t.at[c]], tbuf, sems[b]).wait()

        for g in range(GPC):
            off = c * CHUNK + g * L
            rows = g * L + lanes
            hh = (idx_h[pl.ds(off, L)] & 1) * HIDDEN
            rh = (idx_r[pl.ds(off, L)] & 1) * HIDDEN
            th = (idx_t[pl.ds(off, L)] & 1) * HIDDEN

            def col_body(j, acc):
                h = plsc.load_gather(hbuf, [rows, hh + j])
                r = plsc.load_gather(rbuf, [rows, rh + j])
                t = plsc.load_gather(tbuf, [rows, th + j])
                return acc + jnp.abs(h + r - t)

            acc = lax.fori_loop(0, HIDDEN, col_body,
                                jnp.zeros((L,), jnp.float32))
            scores[pl.ds(off, L)] = -acc

    fire(0, 0)
    for c in range(NCHUNK):
        if c + 1 < NCHUNK:
            fire(c + 1, (c + 1) % NBUF)
        drain_and_compute(c, c % NBUF)

    pltpu.sync_copy(scores, out_hbm.at[pl.ds(base, BPW)])


def kernel(head_index, rel_type, tail_index, node_emb, rel_emb):
    nodep = node_emb.reshape(NUM_NODES // 2, PW)
    relp = rel_emb.reshape(NUM_RELATIONS // 2, PW)
    return _kge_score_sc(
        head_index.astype(jnp.int32),
        rel_type.astype(jnp.int32),
        tail_index.astype(jnp.int32),
        nodep,
        relp,
    )


# speedup vs baseline: 1.5350x; 1.5350x over previous
import functools

import jax
import jax.numpy as jnp
from jax import lax
from jax.experimental import pallas as pl
from jax.experimental.pallas import tpu as pltpu
from jax.experimental.pallas import tpu_sc as plsc

BATCH = 16384
HIDDEN = 64
NUM_RELATIONS = 1000
SUB = 8
L = 16

_info = plsc.get_sparse_core_info()
NC, NS = _info.num_cores, _info.num_subcores
NW = NC * NS
BPW = BATCH // NW
CH = 16
NCHUNK = BPW // CH
NBUF = 4
RSLABS = NUM_RELATIONS // SUB
RSTAGE = 16

_mesh = plsc.VectorSubcoreMesh(core_axis_name="c", subcore_axis_name="s")

_row_bufs = [pltpu.VMEM((CH, HIDDEN), jnp.float32)
             for _ in range(2 * NBUF)]


@functools.partial(
    pl.kernel,
    mesh=_mesh,
    out_type=jax.ShapeDtypeStruct((BATCH,), jnp.float32),
    compiler_params=pltpu.CompilerParams(needs_layout_passes=False),
    scratch_types=[
        pltpu.VMEM((BPW,), jnp.int32),
        pltpu.VMEM((BPW,), jnp.int32),
        pltpu.VMEM((BPW,), jnp.int32),
        pltpu.VMEM((BPW,), jnp.float32),
        pltpu.VMEM((RSTAGE, SUB, HIDDEN), jnp.float32),
        pltpu.VMEM((NUM_RELATIONS * HIDDEN,), jnp.float32),
        pltpu.SemaphoreType.DMA,
    ] + _row_bufs + [pltpu.SemaphoreType.DMA for _ in range(NBUF)],
)
def _kge_score_sc(head_hbm, rel_hbm, tail_hbm, node_hbm, rel3d_hbm, out_hbm,
                  idx_h, idx_r, idx_t, scores, rstage, rcache, sem_r,
                  *bufs_and_sems):
    bufs = [bufs_and_sems[2 * b:2 * b + 2] for b in range(NBUF)]
    sems = bufs_and_sems[2 * NBUF:]

    wid = lax.axis_index("s") * NC + lax.axis_index("c")
    base = wid * BPW

    pltpu.sync_copy(head_hbm.at[pl.ds(base, BPW)], idx_h)
    pltpu.sync_copy(rel_hbm.at[pl.ds(base, BPW)], idx_r)
    pltpu.sync_copy(tail_hbm.at[pl.ds(base, BPW)], idx_t)

    for s0 in range(0, RSLABS, RSTAGE):
        ns = min(RSTAGE, RSLABS - s0)
        pltpu.async_copy(rel3d_hbm.at[pl.ds(s0, ns)], rstage.at[pl.ds(0, ns)],
                         sem_r).wait()

        def compact_body(i, carry, s0=s0):
            s = i >> 3
            r = i & 7
            rowbase = (s0 * SUB + i) * HIDDEN
            for m in range(HIDDEN // L):
                rcache[pl.ds(rowbase + m * L, L)] = (
                    rstage[s, r, pl.ds(m * L, L)])
            return carry

        lax.fori_loop(0, ns * SUB, compact_body, 0)

    lanes = lax.iota(jnp.int32, L)

    def fire(chunk, b):
        off = pl.multiple_of(chunk * CH, CH)
        ihv = idx_h[pl.ds(off, CH)]
        itv = idx_t[pl.ds(off, CH)]
        hbuf, tbuf = bufs[b]
        for k in range(CH):
            dst = pl.ds(k, 1)
            pltpu.async_copy(node_hbm.at[pl.ds(ihv[k], 1), :],
                             hbuf.at[dst], sems[b])
            pltpu.async_copy(node_hbm.at[pl.ds(itv[k], 1), :],
                             tbuf.at[dst], sems[b])

    def drain_and_compute(chunk, b):
        off = pl.multiple_of(chunk * CH, CH)
        hbuf, tbuf = bufs[b]
        for buf in (hbuf, tbuf):
            pltpu.make_async_copy(node_hbm.at[pl.ds(0, CH), :], buf,
                                  sems[b]).wait()
        rbase = idx_r[pl.ds(off, CH)] * HIDDEN

        def col_body(j, acc):
            cj = jnp.full((L,), j, dtype=jnp.int32)
            h = plsc.load_gather(hbuf, [lanes, cj])
            r = plsc.load_gather(rcache, [rbase + j])
            t = plsc.load_gather(tbuf, [lanes, cj])
            return acc + jnp.abs(h + r - t)

        acc = lax.fori_loop(0, HIDDEN, col_body, jnp.zeros((L,), jnp.float32))
        scores[pl.ds(off, CH)] = -acc

    for b in range(NBUF - 1):
        fire(b, b)

    def ring_body(c, carry):
        for b in range(NBUF):
            nxt = c + b + (NBUF - 1)

            @pl.when(nxt < NCHUNK)
            def _():
                fire(nxt, (b + NBUF - 1) % NBUF)

            drain_and_compute(c + b, b)
        return carry

    lax.fori_loop(0, NCHUNK // NBUF, lambda i, cy: ring_body(i * NBUF, cy), 0)

    pltpu.sync_copy(scores, out_hbm.at[pl.ds(base, BPW)])


def kernel(head_index, rel_type, tail_index, node_emb, rel_emb):
    rel3d = rel_emb.reshape(RSLABS, SUB, HIDDEN)
    return _kge_score_sc(
        head_index.astype(jnp.int32),
        rel_type.astype(jnp.int32),
        tail_index.astype(jnp.int32),
        node_emb,
        rel3d,
    )


# speedup vs baseline: 1.5381x; 1.0020x over previous
import functools

import jax
import jax.numpy as jnp
from jax import lax
from jax.experimental import pallas as pl
from jax.experimental.pallas import tpu as pltpu
from jax.experimental.pallas import tpu_sc as plsc

BATCH = 16384
HIDDEN = 64
NUM_RELATIONS = 1000
SUB = 8
L = 16

_info = plsc.get_sparse_core_info()
NC, NS = _info.num_cores, _info.num_subcores
NW = NC * NS
BPW = BATCH // NW
CH = 16
NCHUNK = BPW // CH
NBUF = 4
RSLABS = NUM_RELATIONS // SUB
RSTAGE = 16

_mesh = plsc.VectorSubcoreMesh(core_axis_name="c", subcore_axis_name="s")


@functools.partial(
    pl.kernel,
    mesh=_mesh,
    out_type=jax.ShapeDtypeStruct((BATCH,), jnp.float32),
    compiler_params=pltpu.CompilerParams(needs_layout_passes=False),
    scratch_types=[
        pltpu.VMEM((3 * BPW,), jnp.int32),
        pltpu.VMEM((BPW,), jnp.float32),
        pltpu.VMEM((RSTAGE, SUB, HIDDEN), jnp.float32),
        pltpu.VMEM((NUM_RELATIONS * HIDDEN,), jnp.float32),
        pltpu.VMEM((NBUF, CH, HIDDEN), jnp.float32),
        pltpu.VMEM((NBUF, CH, HIDDEN), jnp.float32),
        pltpu.SemaphoreType.DMA((NBUF + 1,)),
    ],
)
def _kge_score_sc(head_hbm, rel_hbm, tail_hbm, node_hbm, rel3d_hbm, out_hbm,
                  idx3, scores, rstage, rcache, hbufs, tbufs, sems):
    wid = lax.axis_index("s") * NC + lax.axis_index("c")
    base = wid * BPW

    pltpu.sync_copy(head_hbm.at[pl.ds(base, BPW)], idx3.at[pl.ds(0, BPW)])
    pltpu.sync_copy(rel_hbm.at[pl.ds(base, BPW)], idx3.at[pl.ds(BPW, BPW)])
    pltpu.sync_copy(tail_hbm.at[pl.ds(base, BPW)],
                    idx3.at[pl.ds(2 * BPW, BPW)])

    for s0 in range(0, RSLABS, RSTAGE):
        ns = min(RSTAGE, RSLABS - s0)
        pltpu.async_copy(rel3d_hbm.at[pl.ds(s0, ns)], rstage.at[pl.ds(0, ns)],
                         sems.at[NBUF]).wait()

        def compact_body(i, carry, s0=s0):
            s = i >> 3
            r = i & 7
            rowbase = (s0 * SUB + i) * HIDDEN
            for m in range(HIDDEN // L):
                rcache[pl.ds(rowbase + m * L, L)] = (
                    rstage[s, r, pl.ds(m * L, L)])
            return carry

        lax.fori_loop(0, ns * SUB, compact_body, 0)

    lanes = lax.iota(jnp.int32, L)

    def fire(chunk, b):
        off = pl.multiple_of(chunk * CH, CH)
        ihv = idx3[pl.ds(off, CH)]
        itv = idx3[pl.ds(2 * BPW + off, CH)]
        for k in range(CH):
            dst = pl.ds(k, 1)
            pltpu.async_copy(node_hbm.at[pl.ds(ihv[k], 1), :],
                             hbufs.at[b, dst, :], sems.at[b])
            pltpu.async_copy(node_hbm.at[pl.ds(itv[k], 1), :],
                             tbufs.at[b, dst, :], sems.at[b])

    def drain_and_compute(chunk, b):
        off = pl.multiple_of(chunk * CH, CH)
        for bufs in (hbufs, tbufs):
            pltpu.make_async_copy(node_hbm.at[pl.ds(0, CH), :], bufs.at[b],
                                  sems.at[b]).wait()
        rbase = idx3[pl.ds(BPW + off, CH)] * HIDDEN
        bsel = jnp.full((L,), b, dtype=jnp.int32)

        def col_body(j, acc):
            cj = jnp.full((L,), j, dtype=jnp.int32)
            h = plsc.load_gather(hbufs, [bsel, lanes, cj])
            r = plsc.load_gather(rcache, [rbase + j])
            t = plsc.load_gather(tbufs, [bsel, lanes, cj])
            return acc + jnp.abs(h + r - t)

        acc = lax.fori_loop(0, HIDDEN, col_body, jnp.zeros((L,), jnp.float32))
        scores[pl.ds(off, CH)] = -acc

    for b in range(NBUF - 1):
        fire(b, b)

    def ring_body(c, carry):
        for b in range(NBUF):
            nxt = c + b + (NBUF - 1)

            @pl.when(nxt < NCHUNK)
            def _():
                fire(nxt, (b + NBUF - 1) % NBUF)

            drain_and_compute(c + b, b)
        return carry

    lax.fori_loop(0, NCHUNK // NBUF, lambda i, cy: ring_body(i * NBUF, cy), 0)

    pltpu.sync_copy(scores, out_hbm.at[pl.ds(base, BPW)])


def kernel(head_index, rel_type, tail_index, node_emb, rel_emb):
    rel3d = rel_emb.reshape(RSLABS, SUB, HIDDEN)
    return _kge_score_sc(
        head_index.astype(jnp.int32),
        rel_type.astype(jnp.int32),
        tail_index.astype(jnp.int32),
        node_emb,
        rel3d,
    )


# speedup vs baseline: 1.6287x; 1.0589x over previous
import functools

import jax
import jax.numpy as jnp
from jax import lax
from jax.experimental import pallas as pl
from jax.experimental.pallas import tpu as pltpu
from jax.experimental.pallas import tpu_sc as plsc

BATCH = 16384
HIDDEN = 64
L = 16

_info = plsc.get_sparse_core_info()
NC, NS = _info.num_cores, _info.num_subcores
NW = NC * NS
BPW = BATCH // NW
CH = 16
NCHUNK = BPW // CH
NBUF = 4

_mesh = plsc.VectorSubcoreMesh(core_axis_name="c", subcore_axis_name="s")

_row_bufs = [pltpu.VMEM((CH, HIDDEN), jnp.float32)
             for _ in range(3 * NBUF)]


@functools.partial(
    pl.kernel,
    mesh=_mesh,
    out_type=jax.ShapeDtypeStruct((BATCH,), jnp.float32),
    compiler_params=pltpu.CompilerParams(needs_layout_passes=False),
    scratch_types=[
        pltpu.VMEM((BPW,), jnp.int32),
        pltpu.VMEM((BPW,), jnp.int32),
        pltpu.VMEM((BPW,), jnp.int32),
        pltpu.VMEM((BPW,), jnp.float32),
    ] + _row_bufs + [pltpu.SemaphoreType.DMA for _ in range(NBUF)],
)
def _kge_score_sc(head_hbm, rel_hbm, tail_hbm, node_hbm, relemb_hbm, out_hbm,
                  idx_h, idx_r, idx_t, scores, *bufs_and_sems):
    bufs = [bufs_and_sems[3 * b:3 * b + 3] for b in range(NBUF)]
    sems = bufs_and_sems[3 * NBUF:]

    wid = lax.axis_index("s") * NC + lax.axis_index("c")
    base = wid * BPW

    pltpu.sync_copy(head_hbm.at[pl.ds(base, BPW)], idx_h)
    pltpu.sync_copy(rel_hbm.at[pl.ds(base, BPW)], idx_r)
    pltpu.sync_copy(tail_hbm.at[pl.ds(base, BPW)], idx_t)

    lanes = lax.iota(jnp.int32, L)

    def fire(chunk, b):
        off = pl.multiple_of(chunk * CH, CH)
        ihv = idx_h[pl.ds(off, CH)]
        irv = idx_r[pl.ds(off, CH)]
        itv = idx_t[pl.ds(off, CH)]
        hbuf, rbuf, tbuf = bufs[b]
        for k in range(CH):
            dst = pl.ds(k, 1)
            pltpu.async_copy(node_hbm.at[pl.ds(ihv[k], 1), :],
                             hbuf.at[dst], sems[b])
            pltpu.async_copy(relemb_hbm.at[pl.ds(irv[k], 1), :],
                             rbuf.at[dst], sems[b])
            pltpu.async_copy(node_hbm.at[pl.ds(itv[k], 1), :],
                             tbuf.at[dst], sems[b])

    def drain_and_compute(chunk, b):
        hbuf, rbuf, tbuf = bufs[b]
        for buf in (hbuf, rbuf, tbuf):
            pltpu.make_async_copy(node_hbm.at[pl.ds(0, CH), :], buf,
                                  sems[b]).wait()

        def col_body(j, acc):
            cj = jnp.full((L,), j, dtype=jnp.int32)
            h = plsc.load_gather(hbuf, [lanes, cj])
            r = plsc.load_gather(rbuf, [lanes, cj])
            t = plsc.load_gather(tbuf, [lanes, cj])
            return acc + jnp.abs(h + r - t)

        acc = lax.fori_loop(0, HIDDEN, col_body, jnp.zeros((L,), jnp.float32))
        scores[pl.ds(pl.multiple_of(chunk * CH, CH), CH)] = -acc

    for b in range(NBUF - 1):
        fire(b, b)

    def ring_body(c, carry):
        for b in range(NBUF):
            nxt = c + b + (NBUF - 1)

            @pl.when(nxt < NCHUNK)
            def _():
                fire(nxt, (b + NBUF - 1) % NBUF)

            drain_and_compute(c + b, b)
        return carry

    lax.fori_loop(0, NCHUNK // NBUF, lambda i, cy: ring_body(i * NBUF, cy), 0)

    pltpu.sync_copy(scores, out_hbm.at[pl.ds(base, BPW)])


def kernel(head_index, rel_type, tail_index, node_emb, rel_emb):
    return _kge_score_sc(
        head_index.astype(jnp.int32),
        rel_type.astype(jnp.int32),
        tail_index.astype(jnp.int32),
        node_emb,
        rel_emb,
    )


# speedup vs baseline: 1.6306x; 1.0011x over previous
import functools

import jax
import jax.numpy as jnp
from jax import lax
from jax.experimental import pallas as pl
from jax.experimental.pallas import tpu as pltpu
from jax.experimental.pallas import tpu_sc as plsc

BATCH = 16384
HIDDEN = 64
L = 16

_info = plsc.get_sparse_core_info()
NC, NS = _info.num_cores, _info.num_subcores
NW = NC * NS
BPW = BATCH // NW
CH = 32
NCHUNK = BPW // CH
NBUF = 4

_mesh = plsc.VectorSubcoreMesh(core_axis_name="c", subcore_axis_name="s")

_row_bufs = [pltpu.VMEM((CH, HIDDEN), jnp.float32)
             for _ in range(3 * NBUF)]


@functools.partial(
    pl.kernel,
    mesh=_mesh,
    out_type=jax.ShapeDtypeStruct((BATCH,), jnp.float32),
    compiler_params=pltpu.CompilerParams(needs_layout_passes=False),
    scratch_types=[
        pltpu.VMEM((BPW,), jnp.int32),
        pltpu.VMEM((BPW,), jnp.int32),
        pltpu.VMEM((BPW,), jnp.int32),
        pltpu.VMEM((BPW,), jnp.float32),
    ] + _row_bufs + [pltpu.SemaphoreType.DMA for _ in range(NBUF)],
)
def _kge_score_sc(head_hbm, rel_hbm, tail_hbm, node_hbm, relemb_hbm, out_hbm,
                  idx_h, idx_r, idx_t, scores, *bufs_and_sems):
    bufs = [bufs_and_sems[3 * b:3 * b + 3] for b in range(NBUF)]
    sems = bufs_and_sems[3 * NBUF:]

    wid = lax.axis_index("s") * NC + lax.axis_index("c")
    base = wid * BPW

    pltpu.sync_copy(head_hbm.at[pl.ds(base, BPW)], idx_h)
    pltpu.sync_copy(rel_hbm.at[pl.ds(base, BPW)], idx_r)
    pltpu.sync_copy(tail_hbm.at[pl.ds(base, BPW)], idx_t)

    lanes = lax.iota(jnp.int32, L)

    def fire(chunk, b):
        off = pl.multiple_of(chunk * CH, CH)
        hbuf, rbuf, tbuf = bufs[b]
        for g in range(CH // L):
            ihv = idx_h[pl.ds(off + g * L, L)]
            irv = idx_r[pl.ds(off + g * L, L)]
            itv = idx_t[pl.ds(off + g * L, L)]
            for k in range(L):
                dst = pl.ds(g * L + k, 1)
                pltpu.async_copy(node_hbm.at[pl.ds(ihv[k], 1), :],
                                 hbuf.at[dst], sems[b])
                pltpu.async_copy(relemb_hbm.at[pl.ds(irv[k], 1), :],
                                 rbuf.at[dst], sems[b])
                pltpu.async_copy(node_hbm.at[pl.ds(itv[k], 1), :],
                                 tbuf.at[dst], sems[b])

    def drain_and_compute(chunk, b):
        hbuf, rbuf, tbuf = bufs[b]
        for buf in (hbuf, rbuf, tbuf):
            pltpu.make_async_copy(node_hbm.at[pl.ds(0, CH), :], buf,
                                  sems[b]).wait()

        for g in range(CH // L):
            rows = g * L + lanes

            def col_body(j, acc, rows=rows):
                cj = jnp.full((L,), j, dtype=jnp.int32)
                h = plsc.load_gather(hbuf, [rows, cj])
                r = plsc.load_gather(rbuf, [rows, cj])
                t = plsc.load_gather(tbuf, [rows, cj])
                return acc + jnp.abs(h + r - t)

            acc = lax.fori_loop(0, HIDDEN, col_body,
                                jnp.zeros((L,), jnp.float32), unroll=2)
            scores[pl.ds(pl.multiple_of(chunk * CH + g * L, L), L)] = -acc

    for b in range(NBUF - 1):
        fire(b, b)

    def ring_body(c, carry):
        for b in range(NBUF):
            nxt = c + b + (NBUF - 1)

            @pl.when(nxt < NCHUNK)
            def _():
                fire(nxt, (b + NBUF - 1) % NBUF)

            drain_and_compute(c + b, b)
        return carry

    lax.fori_loop(0, NCHUNK // NBUF, lambda i, cy: ring_body(i * NBUF, cy), 0)

    pltpu.sync_copy(scores, out_hbm.at[pl.ds(base, BPW)])


def kernel(head_index, rel_type, tail_index, node_emb, rel_emb):
    return _kge_score_sc(
        head_index.astype(jnp.int32),
        rel_type.astype(jnp.int32),
        tail_index.astype(jnp.int32),
        node_emb,
        rel_emb,
    )
